# Initial kernel scaffold; baseline (speedup 1.0000x reference)
import functools

import jax
import jax.numpy as jnp
from jax import lax
from jax.experimental import pallas as pl
from jax.experimental.pallas import tpu as pltpu
from jax.experimental.pallas import tpu_sc as plsc

NC = 2
NS = 16
LANES = 16
BLK = 128

_ROT_IDX = (1, 2, 0, 4, 5, 3, 7, 8, 6, 10, 11, 9, 12, 13, 14, 15)


def _rot(v):
    idx = jnp.array(_ROT_IDX, dtype=jnp.int32)
    dn = lax.GatherDimensionNumbers(
        offset_dims=(), collapsed_slice_dims=(0,), start_index_map=(0,))
    return lax.gather(v, idx[:, None], dn, (1,),
                      mode=lax.GatherScatterMode.PROMISE_IN_BOUNDS)


def _sc_accumulate(verts16, idx3, v_pad, blocks_per_worker):
    rows_per_sub = v_pad // NS
    zchunks = rows_per_sub // BLK
    mesh = plsc.VectorSubcoreMesh(core_axis_name="c", subcore_axis_name="s")

    @functools.partial(
        pl.kernel,
        out_type=jax.ShapeDtypeStruct((NC, v_pad, LANES), jnp.float32),
        mesh=mesh,
        scratch_types=[
            pltpu.VMEM_SHARED((v_pad, LANES), jnp.float32),
            pltpu.VMEM((3, BLK), jnp.int32),
            pltpu.VMEM((BLK, LANES), jnp.float32),
            pltpu.VMEM((BLK, LANES), jnp.float32),
            pltpu.VMEM((BLK, LANES), jnp.float32),
            pltpu.VMEM((BLK, LANES), jnp.float32),
        ],
    )
    def k(verts_hbm, idx_hbm, out_hbm, acc, idxv, v0b, v1b, v2b, nb):
        c = lax.axis_index("c")
        s = lax.axis_index("s")

        @pl.loop(0, BLK)
        def _(r):
            nb[r] = jnp.zeros((LANES,), jnp.float32)

        base_rows = s * rows_per_sub

        @pl.loop(0, zchunks)
        def _(z):
            pltpu.sync_copy(nb, acc.at[pl.ds(base_rows + z * BLK, BLK)])

        plsc.subcore_barrier()

        wid = c * NS + s

        @pl.loop(0, blocks_per_worker)
        def _(j):
            fbase = (wid * blocks_per_worker + j) * BLK
            pltpu.sync_copy(idx_hbm.at[0, pl.ds(fbase, BLK)], idxv.at[0])
            pltpu.sync_copy(idx_hbm.at[1, pl.ds(fbase, BLK)], idxv.at[1])
            pltpu.sync_copy(idx_hbm.at[2, pl.ds(fbase, BLK)], idxv.at[2])
            pltpu.sync_copy(verts_hbm.at[idxv.at[0]], v0b)
            pltpu.sync_copy(verts_hbm.at[idxv.at[1]], v1b)
            pltpu.sync_copy(verts_hbm.at[idxv.at[2]], v2b)

            @pl.loop(0, BLK)
            def _(r):
                v0 = v0b[r]
                v1 = v1b[r]
                v2 = v2b[r]
                a = v1 - v0
                b = v2 - v0
                nb[r] = a * _rot(b) - _rot(a) * b

            pltpu.sync_copy(nb, acc.at[idxv.at[0]], add=True)
            pltpu.sync_copy(nb, acc.at[idxv.at[1]], add=True)
            pltpu.sync_copy(nb, acc.at[idxv.at[2]], add=True)

        plsc.subcore_barrier()
        pltpu.sync_copy(acc.at[pl.ds(base_rows, rows_per_sub)],
                        out_hbm.at[c, pl.ds(base_rows, rows_per_sub)])

    return k(verts16, idx3)


def _tc_finalize(pt, v_pad):
    cb = 1024

    def body(p_ref, o_ref):
        n = p_ref[0] + p_ref[1]
        fallback = jnp.array([[1.0], [0.0], [0.0]], dtype=jnp.float32)
        for t in range(4):
            nt = n[3 * t:3 * t + 3, :]
            d = jnp.sum(nt * nt, axis=0, keepdims=True)
            m = d > 1e-20
            nv = jnp.where(m, nt, fallback)
            dv = jnp.where(m, d, 1.0)
            o_ref[3 * t:3 * t + 3, :] = nv * lax.rsqrt(jnp.maximum(dv, 1e-20))
        o_ref[12:16, :] = jnp.zeros((4, cb), jnp.float32)

    return pl.pallas_call(
        body,
        grid=(v_pad // cb,),
        in_specs=[pl.BlockSpec((NC, 16, cb), lambda i: (0, 0, i))],
        out_specs=pl.BlockSpec((16, cb), lambda i: (0, i)),
        out_shape=jax.ShapeDtypeStruct((16, v_pad), jnp.float32),
    )(pt)


def kernel(verts, faces):
    nb_batch, nv, _ = verts.shape
    nf = faces.shape[0]
    assert nb_batch == 4

    workers = NC * NS
    blocks_per_worker = -(-nf // (workers * BLK))
    f_pad = workers * blocks_per_worker * BLK
    rows_per_sub = -(-(nv + 1) // (NS * BLK)) * BLK
    v_pad = NS * rows_per_sub

    verts = verts.astype(jnp.float32)
    v16 = jnp.zeros((v_pad, LANES), jnp.float32)
    v16 = v16.at[:nv, :12].set(verts.transpose(1, 0, 2).reshape(nv, 12))
    idx3 = jnp.pad(faces.astype(jnp.int32).T, ((0, 0), (0, f_pad - nf)),
                   constant_values=nv)

    partial_acc = _sc_accumulate(v16, idx3, v_pad, blocks_per_worker)
    out16 = _tc_finalize(partial_acc.transpose(0, 2, 1), v_pad)

    perm = jnp.array([1, 2, 0, 4, 5, 3, 7, 8, 6, 10, 11, 9], dtype=jnp.int32)
    comp = out16[perm, :nv]
    return comp.reshape(4, 3, nv).transpose(0, 2, 1)


# speedup vs baseline: 212.8363x; 212.8363x over previous
import functools

import jax
import jax.numpy as jnp
from jax import lax
from jax.experimental import pallas as pl
from jax.experimental.pallas import tpu as pltpu
from jax.experimental.pallas import tpu_sc as plsc

NC = 2
NS = 16
LANES = 16
BLK = 128

def _rot_idx():
    i = lax.iota(jnp.int32, LANES)
    m = lax.rem(i, 3)
    return jnp.where(i < 12, jnp.where(m == 2, i - 2, i + 1), i)


def _rot(v, idx):
    dn = lax.GatherDimensionNumbers(
        offset_dims=(), collapsed_slice_dims=(0,), start_index_map=(0,))
    return lax.gather(v, idx[:, None], dn, (1,),
                      mode=lax.GatherScatterMode.PROMISE_IN_BOUNDS)


def _sc_accumulate(verts16, idx3, v_pad, blocks_per_worker, f_pad):
    rows_per_sub = v_pad // NS
    zchunks = rows_per_sub // BLK
    mesh = plsc.VectorSubcoreMesh(core_axis_name="c", subcore_axis_name="s")

    @functools.partial(
        pl.kernel,
        out_type=jax.ShapeDtypeStruct((NC, v_pad, LANES), jnp.float32),
        mesh=mesh,
        compiler_params=pltpu.CompilerParams(use_tc_tiling_on_sc=False),
        scratch_types=[
            pltpu.VMEM_SHARED((v_pad, LANES), jnp.float32),
            pltpu.VMEM((3, BLK), jnp.int32),
            pltpu.VMEM((BLK, LANES), jnp.float32),
            pltpu.VMEM((BLK, LANES), jnp.float32),
            pltpu.VMEM((BLK, LANES), jnp.float32),
            pltpu.VMEM((BLK, LANES), jnp.float32),
        ],
    )
    def k(verts_hbm, idx_hbm, out_hbm, acc, idxv, v0b, v1b, v2b, nb):
        c = lax.axis_index("c")
        s = lax.axis_index("s")
        rot = _rot_idx()

        @pl.loop(0, BLK)
        def _(r):
            nb[r] = jnp.zeros((LANES,), jnp.float32)

        base_rows = s * rows_per_sub

        @pl.loop(0, zchunks)
        def _(z):
            pltpu.sync_copy(nb, acc.at[pl.ds(base_rows + z * BLK, BLK)])

        plsc.subcore_barrier()

        wid = c * NS + s

        @pl.loop(0, blocks_per_worker)
        def _(j):
            fbase = (wid * blocks_per_worker + j) * BLK
            pltpu.sync_copy(idx_hbm.at[pl.ds(fbase, BLK)], idxv.at[0])
            pltpu.sync_copy(idx_hbm.at[pl.ds(f_pad + fbase, BLK)], idxv.at[1])
            pltpu.sync_copy(idx_hbm.at[pl.ds(2 * f_pad + fbase, BLK)], idxv.at[2])
            pltpu.sync_copy(verts_hbm.at[idxv.at[0]], v0b)
            pltpu.sync_copy(verts_hbm.at[idxv.at[1]], v1b)
            pltpu.sync_copy(verts_hbm.at[idxv.at[2]], v2b)

            @pl.loop(0, BLK)
            def _(r):
                v0 = v0b[r]
                v1 = v1b[r]
                v2 = v2b[r]
                a = v1 - v0
                b = v2 - v0
                nb[r] = a * _rot(b, rot) - _rot(a, rot) * b

            pltpu.sync_copy(nb, acc.at[idxv.at[0]], add=True)
            pltpu.sync_copy(nb, acc.at[idxv.at[1]], add=True)
            pltpu.sync_copy(nb, acc.at[idxv.at[2]], add=True)

        plsc.subcore_barrier()
        pltpu.sync_copy(acc.at[pl.ds(base_rows, rows_per_sub)],
                        out_hbm.at[c, pl.ds(base_rows, rows_per_sub)])

    return k(verts16, idx3)


def _tc_finalize(pt, v_pad):
    cb = 1024

    def body(p_ref, o_ref):
        n = p_ref[0] + p_ref[1]
        fi = lax.broadcasted_iota(jnp.int32, (3, cb), 0)
        fallback = jnp.where(fi == 0, 1.0, 0.0).astype(jnp.float32)
        for t in range(4):
            nt = n[3 * t:3 * t + 3, :]
            d = jnp.sum(nt * nt, axis=0, keepdims=True)
            m = d > 1e-20
            nv = jnp.where(m, nt, fallback)
            dv = jnp.where(m, d, 1.0)
            o_ref[3 * t:3 * t + 3, :] = nv * lax.rsqrt(jnp.maximum(dv, 1e-20))
        o_ref[12:16, :] = jnp.zeros((4, cb), jnp.float32)

    return pl.pallas_call(
        body,
        grid=(v_pad // cb,),
        in_specs=[pl.BlockSpec((NC, 16, cb), lambda i: (0, 0, i))],
        out_specs=pl.BlockSpec((16, cb), lambda i: (0, i)),
        out_shape=jax.ShapeDtypeStruct((16, v_pad), jnp.float32),
    )(pt)


def kernel(verts, faces):
    nb_batch, nv, _ = verts.shape
    nf = faces.shape[0]
    assert nb_batch == 4

    workers = NC * NS
    blocks_per_worker = -(-nf // (workers * BLK))
    f_pad = workers * blocks_per_worker * BLK
    rows_per_sub = -(-(nv + 1) // (NS * BLK)) * BLK
    v_pad = NS * rows_per_sub

    verts = verts.astype(jnp.float32)
    v16 = jnp.zeros((v_pad, LANES), jnp.float32)
    v16 = v16.at[:nv, :12].set(verts.transpose(1, 0, 2).reshape(nv, 12))
    idx3 = jnp.pad(faces.astype(jnp.int32).T, ((0, 0), (0, f_pad - nf)),
                   constant_values=nv).reshape(-1)

    partial_acc = _sc_accumulate(v16, idx3, v_pad, blocks_per_worker, f_pad)
    out16 = _tc_finalize(partial_acc.transpose(0, 2, 1), v_pad)

    perm = jnp.array([1, 2, 0, 4, 5, 3, 7, 8, 6, 10, 11, 9], dtype=jnp.int32)
    comp = out16[perm, :nv]
    return comp.reshape(4, 3, nv).transpose(0, 2, 1)
